# Initial kernel scaffold; baseline (speedup 1.0000x reference)
import jax
import jax.numpy as jnp
from jax.experimental import pallas as pl


def kernel(feats, segment_ids, Ws, bs, shared_W, shared_b):
    raise NotImplementedError("write your pallas kernel here")


# speedup vs baseline: 3.8571x; 3.8571x over previous
import functools

import jax
import jax.numpy as jnp
from jax import lax
from jax.experimental import pallas as pl
from jax.experimental.pallas import tpu as pltpu
from jax.experimental.pallas import tpu_sc as plsc

N = 160000
D = 512
T = 12
TP = 16
S = 4096
L = 16
NF = D // L

NC = 2
NS = 16
NW = NC * NS
SEG_PER_W = S // NW
OFF_SLICE = 144
CH = 64


_TW_BLK = 2000


def _tw_body(x_ref, w_ref, b_ref, o_ref):
    z = jnp.dot(x_ref[...], w_ref[...], preferred_element_type=jnp.float32)
    o_ref[...] = jax.nn.sigmoid(z + b_ref[...])


def _task_weights(feats, ws_t_pad, bs_pad):
    return pl.pallas_call(
        _tw_body,
        grid=(N // _TW_BLK,),
        in_specs=[
            pl.BlockSpec((_TW_BLK, D), lambda i: (i, 0)),
            pl.BlockSpec((D, TP), lambda i: (0, 0)),
            pl.BlockSpec((1, TP), lambda i: (0, 0)),
        ],
        out_specs=pl.BlockSpec((_TW_BLK, TP), lambda i: (i, 0)),
        out_shape=jax.ShapeDtypeStruct((N, TP), jnp.float32),
    )(feats, ws_t_pad, bs_pad)


_MESH = plsc.VectorSubcoreMesh(core_axis_name="c", subcore_axis_name="s")


@functools.partial(
    pl.kernel,
    out_type=jax.ShapeDtypeStruct((S * T * D,), jnp.float32),
    mesh=_MESH,
    scratch_types=[
        pltpu.VMEM((OFF_SLICE,), jnp.int32),
        pltpu.VMEM((CH * D,), jnp.float32),
        pltpu.VMEM((CH * TP,), jnp.float32),
        pltpu.VMEM((T * D,), jnp.float32),
    ],
)
def _sc_seg_sum(feats_hbm, tw_hbm, off_hbm, out_hbm, off_v, fbuf, wbuf, acc):
    cid = lax.axis_index("c")
    sid = lax.axis_index("s")
    wid = sid * NC + cid
    s0 = wid * SEG_PER_W

    pltpu.sync_copy(off_hbm.at[pl.ds(s0, OFF_SLICE)], off_v)

    zero16 = jnp.zeros((L,), jnp.float32)

    def zero_frag(f, _):
        for t in range(T):
            acc[pl.ds(t * D + f * L, L)] = zero16
        return 0

    def seg_body(sl, _):
        ovec = off_v[pl.ds(sl, L)]
        r0 = ovec[0]
        r1 = ovec[1]
        n = r1 - r0
        lax.fori_loop(0, NF, zero_frag, 0, unroll=False)

        def chunk_body(k, _):
            start = r0 + k * CH
            cs = jnp.minimum(start, N - CH)
            dlt = start - cs
            pltpu.sync_copy(feats_hbm.at[pl.ds(cs * D, CH * D)], fbuf)
            pltpu.sync_copy(tw_hbm.at[pl.ds(cs * TP, CH * TP)], wbuf)
            m = jnp.minimum(n - k * CH, CH)

            def row_body(i, _):
                ri = dlt + i
                wvec = wbuf[pl.ds(ri * TP, TP)]
                w = [wvec[t] for t in range(T)]

                def frag_body(f, _):
                    v = fbuf[pl.ds(ri * D + f * L, L)]
                    for t in range(T):
                        acc[pl.ds(t * D + f * L, L)] += w[t] * v
                    return 0

                lax.fori_loop(0, NF, frag_body, 0, unroll=False)
                return 0

            lax.fori_loop(0, m, row_body, 0, unroll=False)
            return 0

        nch = (n + CH - 1) // CH
        lax.fori_loop(0, nch, chunk_body, 0, unroll=False)
        pltpu.sync_copy(acc, out_hbm.at[pl.ds((s0 + sl) * (T * D), T * D)])
        return 0

    lax.fori_loop(0, SEG_PER_W, seg_body, 0, unroll=False)


def kernel(feats, segment_ids, Ws, bs, shared_W, shared_b):
    ws_t_pad = jnp.zeros((D, TP), jnp.float32).at[:, :T].set(Ws.T)
    bs_pad = jnp.zeros((1, TP), jnp.float32).at[0, :T].set(bs)
    tw = _task_weights(feats, ws_t_pad, bs_pad)

    offs = jnp.searchsorted(segment_ids, jnp.arange(S + 1, dtype=jnp.int32),
                            side="left").astype(jnp.int32)
    offs = jnp.concatenate(
        [offs, jnp.full(((NW - 1) * SEG_PER_W + OFF_SLICE) - (S + 1), N,
                        dtype=jnp.int32)])

    out = _sc_seg_sum(feats.reshape(-1), tw.reshape(-1), offs)
    return jnp.transpose(out.reshape(S, T, D), (1, 0, 2))


# speedup vs baseline: 6.7356x; 1.7463x over previous
import functools

import jax
import jax.numpy as jnp
from jax import lax
from jax.experimental import pallas as pl
from jax.experimental.pallas import tpu as pltpu
from jax.experimental.pallas import tpu_sc as plsc

N = 160000
D = 512
T = 12
TP = 16
S = 4096
L = 16
NF = D // L

NC = 2
NS = 16
NW = NC * NS
SEG_PER_W = S // NW
OFF_SLICE = 144
CH = 64
RB = 4


_TW_BLK = 2000


def _tw_body(x_ref, w_ref, b_ref, o_ref):
    z = jnp.dot(x_ref[...], w_ref[...], preferred_element_type=jnp.float32)
    o_ref[...] = jax.nn.sigmoid(z + b_ref[...])


def _task_weights(feats, ws_t_pad, bs_pad):
    return pl.pallas_call(
        _tw_body,
        grid=(N // _TW_BLK,),
        in_specs=[
            pl.BlockSpec((_TW_BLK, D), lambda i: (i, 0)),
            pl.BlockSpec((D, TP), lambda i: (0, 0)),
            pl.BlockSpec((1, TP), lambda i: (0, 0)),
        ],
        out_specs=pl.BlockSpec((_TW_BLK, TP), lambda i: (i, 0)),
        out_shape=jax.ShapeDtypeStruct((N, TP), jnp.float32),
    )(feats, ws_t_pad, bs_pad)


_MESH = plsc.VectorSubcoreMesh(core_axis_name="c", subcore_axis_name="s")


@functools.partial(
    pl.kernel,
    out_type=jax.ShapeDtypeStruct((S * T * D,), jnp.float32),
    mesh=_MESH,
    scratch_types=[
        pltpu.VMEM((OFF_SLICE,), jnp.int32),
        pltpu.VMEM((CH * D,), jnp.float32),
        pltpu.VMEM((CH * TP,), jnp.float32),
        pltpu.VMEM((T * D,), jnp.float32),
    ],
)
def _sc_seg_sum(feats_hbm, tw_hbm, off_hbm, out_hbm, off_v, fbuf, wbuf, acc):
    cid = lax.axis_index("c")
    sid = lax.axis_index("s")
    wid = sid * NC + cid
    s0 = wid * SEG_PER_W

    pltpu.sync_copy(off_hbm.at[pl.ds(s0, OFF_SLICE)], off_v)

    zero16 = jnp.zeros((L,), jnp.float32)

    def zero_frag(f, _):
        for t in range(T):
            acc[pl.ds(t * D + f * L, L)] = zero16
        return 0

    def seg_body(sl, _):
        ovec = off_v[pl.ds(sl, L)]
        r0 = ovec[0]
        r1 = ovec[1]
        n = r1 - r0
        lax.fori_loop(0, NF, zero_frag, 0, unroll=False)

        def chunk_body(k, _):
            start = r0 + k * CH
            cs = jnp.minimum(start, N - CH)
            dlt = start - cs
            pltpu.sync_copy(feats_hbm.at[pl.ds(cs * D, CH * D)], fbuf)
            pltpu.sync_copy(tw_hbm.at[pl.ds(cs * TP, CH * TP)], wbuf)
            m = jnp.minimum(n - k * CH, CH)
            nb = m // RB

            def batch_body(b, _):
                ri0 = dlt + b * RB
                wv = [wbuf[pl.ds((ri0 + r) * TP, TP)] for r in range(RB)]
                w = [[wv[r][t] for t in range(T)] for r in range(RB)]

                def frag_body(f, _):
                    v = [fbuf[pl.ds((ri0 + r) * D + f * L, L)]
                         for r in range(RB)]
                    for t in range(T):
                        p = w[0][t] * v[0]
                        for r in range(1, RB):
                            p = p + w[r][t] * v[r]
                        plsc.addupdate(acc.at[pl.ds(t * D + f * L, L)], p)
                    return 0

                lax.fori_loop(0, NF, frag_body, 0, unroll=False)
                return 0

            lax.fori_loop(0, nb, batch_body, 0, unroll=False)

            def row_body(i, _):
                ri = dlt + i
                wvec = wbuf[pl.ds(ri * TP, TP)]
                w = [wvec[t] for t in range(T)]

                def frag_body(f, _):
                    v = fbuf[pl.ds(ri * D + f * L, L)]
                    for t in range(T):
                        plsc.addupdate(acc.at[pl.ds(t * D + f * L, L)],
                                       w[t] * v)
                    return 0

                lax.fori_loop(0, NF, frag_body, 0, unroll=False)
                return 0

            lax.fori_loop(nb * RB, m, row_body, 0, unroll=False)
            return 0

        nch = (n + CH - 1) // CH
        lax.fori_loop(0, nch, chunk_body, 0, unroll=False)
        pltpu.sync_copy(acc, out_hbm.at[pl.ds((s0 + sl) * (T * D), T * D)])
        return 0

    lax.fori_loop(0, SEG_PER_W, seg_body, 0, unroll=False)


def kernel(feats, segment_ids, Ws, bs, shared_W, shared_b):
    ws_t_pad = jnp.zeros((D, TP), jnp.float32).at[:, :T].set(Ws.T)
    bs_pad = jnp.zeros((1, TP), jnp.float32).at[0, :T].set(bs)
    tw = _task_weights(feats, ws_t_pad, bs_pad)

    offs = jnp.searchsorted(segment_ids, jnp.arange(S + 1, dtype=jnp.int32),
                            side="left").astype(jnp.int32)
    offs = jnp.concatenate(
        [offs, jnp.full(((NW - 1) * SEG_PER_W + OFF_SLICE) - (S + 1), N,
                        dtype=jnp.int32)])

    out = _sc_seg_sum(feats.reshape(-1), tw.reshape(-1), offs)
    return jnp.transpose(out.reshape(S, T, D), (1, 0, 2))
